# baseline (speedup 1.0000x reference)
import jax
import jax.numpy as jnp
from jax.experimental import pallas as pl


_BATCH = 16384
_K = 64
_BR = 2048


def _body(gu_ref, gi_ref, out_ref):
    out_ref[...] = jnp.sum(gu_ref[...] * gi_ref[...], axis=1)


def kernel(gu, gi):
    grid = (_BATCH // _BR,)
    out = pl.pallas_call(
        _body,
        grid=grid,
        in_specs=[
            pl.BlockSpec((_BR, _K), lambda i: (i, 0)),
            pl.BlockSpec((_BR, _K), lambda i: (i, 0)),
        ],
        out_specs=pl.BlockSpec((_BR,), lambda i: (i,)),
        out_shape=jax.ShapeDtypeStruct((_BATCH,), jnp.float32),
    )(gu, gi)
    return out
